# baseline (speedup 1.0000x reference)
import functools

import jax
import jax.numpy as jnp
from jax import lax
from jax.experimental import pallas as pl
from jax.experimental.pallas import tpu as pltpu
from jax.experimental.pallas import tpu_sc as plsc

IN_CH = 256
HID = 256
NUM_CLASSES = 4


def _sc_neighbor_sum(seq_len, d, n_workers, nc):
    rpw = seq_len // n_workers

    mesh = plsc.VectorSubcoreMesh(core_axis_name="c", subcore_axis_name="s")

    @functools.partial(
        pl.kernel, mesh=mesh,
        out_type=(jax.ShapeDtypeStruct((seq_len, d), jnp.float32),
                  jax.ShapeDtypeStruct((seq_len, d), jnp.float32)),
        scratch_types=[
            pltpu.VMEM((rpw,), jnp.int32),
            pltpu.VMEM((rpw,), jnp.int32),
            pltpu.VMEM((rpw, d), jnp.float32),
            pltpu.VMEM((rpw, d), jnp.float32),
            pltpu.SemaphoreType.DMA,
            pltpu.SemaphoreType.DMA,
        ],
    )
    def sc_nbr(x_hbm, ip_hbm, in_hbm, yp_hbm, yn_hbm, ip_v, in_v, rp_v, rn_v,
               sem, sem2):
        wid = lax.axis_index("s") * nc + lax.axis_index("c")
        base = wid * rpw
        pltpu.sync_copy(ip_hbm.at[pl.ds(base, rpw)], ip_v)
        pltpu.sync_copy(in_hbm.at[pl.ds(base, rpw)], in_v)
        pltpu.async_copy(x_hbm.at[ip_v], rp_v, sem)
        pltpu.async_copy(x_hbm.at[in_v], rn_v, sem2)
        pltpu.make_async_copy(x_hbm.at[ip_v], rp_v, sem).wait()
        pltpu.make_async_copy(x_hbm.at[in_v], rn_v, sem2).wait()
        pltpu.sync_copy(rp_v, yp_hbm.at[pl.ds(base, rpw)])
        pltpu.sync_copy(rn_v, yn_hbm.at[pl.ds(base, rpw)])

    return sc_nbr


def _plain_block_kernel(x_ref, w1_ref, b1_ref, part_ref, *, blk):
    xb = x_ref[...]
    ones = jnp.ones((8, blk), jnp.float32)
    h = jnp.maximum(
        jnp.dot(xb, w1_ref[...], preferred_element_type=jnp.float32)
        + b1_ref[...], 0.0)
    part_ref[...] = jnp.dot(ones, h, preferred_element_type=jnp.float32)


def _chain_final_kernel(x_ref, yp_ref, yn_ref, w1_ref, b1_ref, parts_ref,
                        w2_ref, b2_ref, out_ref, *, blk, seq_len, n_total,
                        n_parts):
    xb = x_ref[...]
    w1 = w1_ref[...]
    b1 = b1_ref[...]
    ones = jnp.ones((8, seq_len), jnp.float32)

    y0 = xb[:seq_len, :] + 0.5 * (yp_ref[...] + yn_ref[...])
    h0 = jnp.maximum(
        jnp.dot(y0, w1, preferred_element_type=jnp.float32) + b1, 0.0)
    total = jnp.dot(ones, h0, preferred_element_type=jnp.float32)

    h1 = jnp.maximum(
        jnp.dot(xb[seq_len:, :], w1, preferred_element_type=jnp.float32)
        + b1, 0.0)
    ones1 = jnp.ones((8, blk - seq_len), jnp.float32)
    total += jnp.dot(ones1, h1, preferred_element_type=jnp.float32)

    for i in range(n_parts):
        total += parts_ref[8 * i:8 * (i + 1), :]

    pooled = total / jnp.float32(n_total)
    out_ref[...] = (jnp.dot(pooled, w2_ref[...],
                            preferred_element_type=jnp.float32)
                    + b2_ref[...])


def kernel(x, W1, b1, W2, b2):
    batch, seq_len, d = x.shape
    xf = x.reshape(-1, d)
    n = xf.shape[0]
    blk = 2 * seq_len
    nblocks = n // blk
    rep = 8

    hid = W1.shape[1]
    ncls = W2.shape[1]
    w2p = jnp.zeros((hid, 128), W2.dtype).at[:, :ncls].set(W2)
    b2p = jnp.zeros((128,), b2.dtype).at[:ncls].set(b2)

    ar = jnp.arange(seq_len, dtype=jnp.int32)
    idx_prev = jnp.abs(ar - 1)
    idx_next = (seq_len - 1) - jnp.abs((seq_len - 2) - ar)

    info = plsc.get_sparse_core_info()
    n_workers = info.num_cores * info.num_subcores
    yp, yn = _sc_neighbor_sum(seq_len, d, n_workers, info.num_cores)(
        xf, idx_prev, idx_next)

    parts = pl.pallas_call(
        functools.partial(_plain_block_kernel, blk=blk),
        grid=(nblocks - 1,),
        in_specs=[
            pl.BlockSpec((blk, d), lambda k: (k + 1, 0)),
            pl.BlockSpec((d, hid), lambda k: (0, 0)),
            pl.BlockSpec((hid,), lambda k: (0,)),
        ],
        out_specs=pl.BlockSpec((rep, hid), lambda k: (k, 0)),
        out_shape=jax.ShapeDtypeStruct(((nblocks - 1) * rep, hid),
                                       jnp.float32),
    )(xf, W1, b1)

    out = pl.pallas_call(
        functools.partial(_chain_final_kernel, blk=blk, seq_len=seq_len,
                          n_total=n, n_parts=nblocks - 1),
        grid=(1,),
        in_specs=[
            pl.BlockSpec((blk, d), lambda k: (0, 0)),
            pl.BlockSpec((seq_len, d), lambda k: (0, 0)),
            pl.BlockSpec((seq_len, d), lambda k: (0, 0)),
            pl.BlockSpec((d, hid), lambda k: (0, 0)),
            pl.BlockSpec((hid,), lambda k: (0,)),
            pl.BlockSpec(((nblocks - 1) * rep, hid), lambda k: (0, 0)),
            pl.BlockSpec((hid, 128), lambda k: (0, 0)),
            pl.BlockSpec((128,), lambda k: (0,)),
        ],
        out_specs=pl.BlockSpec((8, 128), lambda k: (0, 0)),
        out_shape=jax.ShapeDtypeStruct((8, 128), jnp.float32),
    )(xf, yp, yn, W1, b1, parts, w2p, b2p)
    return out[0:1, :ncls]


# speedup vs baseline: 2.0821x; 2.0821x over previous
import functools

import jax
import jax.numpy as jnp
from jax.experimental import pallas as pl
from jax.experimental.pallas import tpu as pltpu

IN_CH = 256
HID = 256
NUM_CLASSES = 4


def _fused_kernel(x_ref, w1_ref, b1_ref, w2_ref, b2_ref, out_ref, acc_ref,
                  *, blk, seq_len, n_total):
    k = pl.program_id(0)
    nblocks = pl.num_programs(0)

    @pl.when(k == 0)
    def _init():
        acc_ref[...] = jnp.zeros_like(acc_ref)

    xb = x_ref[...]
    ones = jnp.ones((8, blk), jnp.float32)

    def accumulate(y):
        h = jnp.maximum(
            jnp.dot(y, w1_ref[...], preferred_element_type=jnp.float32)
            + b1_ref[...], 0.0)
        acc_ref[...] += jnp.dot(ones, h, preferred_element_type=jnp.float32)

    @pl.when(k == 0)
    def _chain_block():
        idx = jax.lax.broadcasted_iota(jnp.int32, xb.shape, 0)
        has_prev = jnp.logical_and(idx > 0, idx < seq_len)
        has_next = idx < (seq_len - 1)
        prev = pltpu.roll(xb, 1, 0)
        nxt = pltpu.roll(xb, blk - 1, 0)
        zero = jnp.zeros_like(xb)
        nbr = jnp.where(has_prev, prev, zero) + jnp.where(has_next, nxt, zero)
        inv_deg = jnp.where(jnp.logical_and(has_prev, has_next), 0.5, 1.0)
        accumulate(xb + nbr * inv_deg)

    @pl.when(k != 0)
    def _plain_block():
        accumulate(xb)

    @pl.when(k == nblocks - 1)
    def _final():
        pooled = acc_ref[...] / jnp.float32(n_total)
        logits = (jnp.dot(pooled, w2_ref[...],
                          preferred_element_type=jnp.float32)
                  + b2_ref[...])
        out_ref[...] = logits


def kernel(x, W1, b1, W2, b2):
    batch, seq_len, d = x.shape
    xf = x.reshape(-1, d)
    n = xf.shape[0]
    blk = 2 * seq_len
    nblocks = n // blk

    hid = W1.shape[1]
    ncls = W2.shape[1]
    w2p = jnp.zeros((hid, 128), W2.dtype).at[:, :ncls].set(W2)
    b2p = jnp.zeros((128,), b2.dtype).at[:ncls].set(b2)

    out = pl.pallas_call(
        functools.partial(_fused_kernel, blk=blk, seq_len=seq_len, n_total=n),
        grid=(nblocks,),
        in_specs=[
            pl.BlockSpec((blk, d), lambda k: (k, 0)),
            pl.BlockSpec((d, hid), lambda k: (0, 0)),
            pl.BlockSpec((hid,), lambda k: (0,)),
            pl.BlockSpec((hid, 128), lambda k: (0, 0)),
            pl.BlockSpec((128,), lambda k: (0,)),
        ],
        out_specs=pl.BlockSpec((8, 128), lambda k: (0, 0)),
        out_shape=jax.ShapeDtypeStruct((8, 128), jnp.float32),
        scratch_shapes=[pltpu.VMEM((8, hid), jnp.float32)],
    )(xf, W1, b1, w2p, b2p)
    return out[0:1, :ncls]
